# baseline (speedup 1.0000x reference)
import functools

import jax
import jax.numpy as jnp
from jax import lax
from jax.experimental import pallas as pl
from jax.experimental.pallas import tpu as pltpu
from jax.experimental.pallas import tpu_sc as plsc

VOCAB = 1000000
EMB = 64
BATCH = 16384
CTX = 20

NC = 2
NS = 16
NW = NC * NS
BPW = BATCH // NW
RC = 64
NCH = BPW // RC
INV_CTX = 1.0 / CTX


def _make_mesh():
    return plsc.VectorSubcoreMesh(
        core_axis_name="c", subcore_axis_name="s", num_cores=NC, num_subcores=NS
    )


_scratch = [
    pltpu.VMEM((BPW * CTX,), jnp.int32),
    pltpu.VMEM((CTX * NCH, RC), jnp.int32),
    pltpu.VMEM((CTX, RC, EMB), jnp.float32),
    pltpu.VMEM((2, RC, EMB), jnp.float32),
    pltpu.SemaphoreType.DMA,
    pltpu.SemaphoreType.DMA,
]


def _cbow_body(x_hbm, table_hbm, out_hbm, idx_flat, idx_t, stg, obuf, gsem, osem):
    wid = lax.axis_index("s") * NC + lax.axis_index("c")
    base = wid * BPW

    pltpu.sync_copy(x_hbm.at[pl.ds(base * CTX, BPW * CTX)], idx_flat)

    iota16 = lax.iota(jnp.int32, 16)
    step = iota16 * CTX

    @pl.loop(0, CTX)
    def _transpose(g):
        for cc in range(NCH):
            for t in range(RC // 16):
                lanes = step + ((cc * RC + t * 16) * CTX + g)
                idx_t[g * NCH + cc, pl.ds(t * 16, 16)] = plsc.load_gather(
                    idx_flat, [lanes]
                )

    for cc in range(NCH):
        for g in range(CTX):
            pltpu.async_copy(
                table_hbm.at[idx_t.at[g * NCH + cc]], stg.at[g], gsem
            )
        for g in range(CTX):
            pltpu.make_async_copy(
                table_hbm.at[idx_t.at[g * NCH + cc]], stg.at[g], gsem
            ).wait()

        ob = obuf.at[cc % 2]
        if cc >= 2:
            pltpu.make_async_copy(
                ob, out_hbm.at[pl.ds(base, RC)], osem
            ).wait()

        @pl.loop(0, RC)
        def _accum(r):
            for v in range(EMB // 16):
                sl = pl.ds(v * 16, 16)
                s = stg[0, r, sl]
                for g in range(1, CTX):
                    s = s + stg[g, r, sl]
                ob[r, sl] = s * INV_CTX

        pltpu.async_copy(ob, out_hbm.at[pl.ds(base + cc * RC, RC)], osem)

    for cc in range(NCH - 2, NCH):
        pltpu.make_async_copy(
            obuf.at[cc % 2], out_hbm.at[pl.ds(base, RC)], osem
        ).wait()


_cbow_sc_cache = []


def _get_cbow_sc():
    if not _cbow_sc_cache:
        _cbow_sc_cache.append(
            pl.kernel(
                _cbow_body,
                mesh=_make_mesh(),
                out_type=jax.ShapeDtypeStruct((BATCH, EMB), jnp.float32),
                scratch_types=_scratch,
                compiler_params=pltpu.CompilerParams(
                    needs_layout_passes=False, use_tc_tiling_on_sc=False
                ),
            )
        )
    return _cbow_sc_cache[0]


def kernel(x, embedding_table):
    return _get_cbow_sc()(x.reshape(BATCH * CTX), embedding_table)
